# baseline (speedup 1.0000x reference)
import functools

import jax
import jax.numpy as jnp
from jax import lax
from jax.experimental import pallas as pl
from jax.experimental.pallas import tpu as pltpu
from jax.experimental.pallas import tpu_sc as plsc

_NC = 2
_NS = 16
_NW = _NC * _NS

_D = 64
_CHUNK = 800


def _make_gather(total, dtype):
    n_per_w = total // _NW
    n_chunks = n_per_w // _CHUNK
    assert n_chunks * _CHUNK == n_per_w and n_chunks >= 3

    mesh = plsc.VectorSubcoreMesh(
        core_axis_name="c", subcore_axis_name="s",
        num_cores=_NC, num_subcores=_NS)

    @functools.partial(
        pl.kernel,
        mesh=mesh,
        compiler_params=pltpu.CompilerParams(use_tc_tiling_on_sc=False),
        out_type=jax.ShapeDtypeStruct((total, _D), dtype),
        scratch_types=[
            pltpu.VMEM((n_per_w,), jnp.int32),
            pltpu.VMEM((_CHUNK, _D), dtype),
            pltpu.VMEM((_CHUNK, _D), dtype),
            pltpu.SemaphoreType.DMA,
            pltpu.SemaphoreType.DMA,
            pltpu.SemaphoreType.DMA,
            pltpu.SemaphoreType.DMA,
        ],
    )
    def gather(table_hbm, idx_hbm, out_hbm, idx_v, rows_a, rows_b,
               gsem_a, gsem_b, wsem_a, wsem_b):
        wid = lax.axis_index("s") * _NC + lax.axis_index("c")
        base = wid * n_per_w
        pltpu.sync_copy(idx_hbm.at[pl.ds(base, n_per_w)], idx_v)
        bufs = (rows_a, rows_b)
        gsems = (gsem_a, gsem_b)
        wsems = (wsem_a, wsem_b)

        def out_slice(j):
            return out_hbm.at[pl.ds(base + j * _CHUNK, _CHUNK)]

        def issue(j):
            pltpu.async_copy(
                table_hbm.at[idx_v.at[pl.ds(j * _CHUNK, _CHUNK)]],
                bufs[j % 2], gsems[j % 2])

        def gwait(j):
            pltpu.make_async_copy(
                table_hbm.at[idx_v.at[pl.ds(j * _CHUNK, _CHUNK)]],
                bufs[j % 2], gsems[j % 2]).wait()

        def wstart(j):
            pltpu.async_copy(bufs[j % 2], out_slice(j), wsems[j % 2])

        def wwait(j):
            pltpu.make_async_copy(bufs[j % 2], out_slice(j),
                                  wsems[j % 2]).wait()

        issue(0)
        for j in range(1, n_chunks):
            if j >= 2:
                wwait(j - 2)
            issue(j)
            gwait(j - 1)
            wstart(j - 1)
        gwait(n_chunks - 1)
        wwait(n_chunks - 2)
        wstart(n_chunks - 1)
        wwait(n_chunks - 1)

    return gather


def kernel(x, index, latents):
    b, h = index.shape
    num_rows, d = latents.shape
    idx_flat = index.reshape(b * h).astype(jnp.int32)
    out = _make_gather(b * h, latents.dtype)(latents, idx_flat)
    return out.reshape(b, h, d)
